# baseline (speedup 1.0000x reference)
import functools

import jax
import jax.numpy as jnp
from jax import lax
from jax.experimental import pallas as pl
from jax.experimental.pallas import tpu as pltpu
from jax.experimental.pallas import tpu_sc as plsc

G0, G1, D = 128, 64, 256
L = 16
NC, NS = 2, 16
NW = NC * NS
QPW = G0 // NW
KD = D // L

_mesh = plsc.VectorSubcoreMesh(core_axis_name="c", subcore_axis_name="s")


@functools.partial(
    pl.kernel,
    mesh=_mesh,
    out_type=jax.ShapeDtypeStruct((G0, G1), jnp.float32),
    scratch_types=[
        pltpu.VMEM((D,), jnp.float32),
        pltpu.VMEM((2, G1, D), jnp.float32),
        pltpu.VMEM((QPW, G1), jnp.float32),
        pltpu.SemaphoreType.DMA,
        pltpu.SemaphoreType.DMA,
    ],
)
def _som_distances(x_hbm, w_hbm, out_hbm, x_v, w_v, o_v, sem0, sem1):
    wid = lax.axis_index("s") * NC + lax.axis_index("c")
    q0 = wid * QPW
    sems = (sem0, sem1)

    pltpu.sync_copy(x_hbm, x_v)
    xs = [x_v[pl.ds(k * L, L)] for k in range(KD)]
    lanes = lax.iota(jnp.int32, L)
    perms = [lanes ^ s for s in (8, 4, 2, 1)]

    pending = [None, None]
    pending[0] = pltpu.async_copy(w_hbm.at[q0], w_v.at[0], sem0)
    for q in range(QPW):
        cur = q & 1
        if q + 1 < QPW:
            nxt = (q + 1) & 1
            pending[nxt] = pltpu.async_copy(
                w_hbm.at[q0 + q + 1], w_v.at[nxt], sems[nxt]
            )
        pending[cur].wait()

        def group_body(g, carry, cur=cur, q=q):
            c0 = g * L
            out_vec = jnp.zeros((L,), jnp.float32)
            for j in range(L):
                acc = jnp.zeros((L,), jnp.float32)
                for k in range(KD):
                    d = w_v[cur, c0 + j, pl.ds(k * L, L)] - xs[k]
                    acc = acc + d * d
                for p in perms:
                    acc = acc + acc.at[p].get(
                        mode="promise_in_bounds", unique_indices=True
                    )
                out_vec = jnp.where(lanes == j, acc, out_vec)
            o_v[q, pl.ds(c0, L)] = out_vec
            return carry

        lax.fori_loop(0, G1 // L, group_body, 0)

    pltpu.sync_copy(o_v, out_hbm.at[pl.ds(q0, QPW)])


def kernel(x, weights):
    return _som_distances(x, weights)


# speedup vs baseline: 1.0697x; 1.0697x over previous
import functools

import jax
import jax.numpy as jnp
from jax import lax
from jax.experimental import pallas as pl
from jax.experimental.pallas import tpu as pltpu
from jax.experimental.pallas import tpu_sc as plsc

G0, G1, D = 128, 64, 256
L = 16
NC, NS = 2, 16
NW = NC * NS
QPW = G0 // NW
KD = D // L

LEAF = (0, 8, 4, 12, 2, 10, 6, 14, 1, 9, 5, 13, 3, 11, 7, 15)

_mesh = plsc.VectorSubcoreMesh(core_axis_name="c", subcore_axis_name="s")


@functools.partial(
    pl.kernel,
    mesh=_mesh,
    out_type=jax.ShapeDtypeStruct((G0, G1), jnp.float32),
    scratch_types=[
        pltpu.VMEM((D,), jnp.float32),
        pltpu.VMEM((QPW, G1, D), jnp.float32),
        pltpu.VMEM((QPW, G1), jnp.float32),
        pltpu.SemaphoreType.DMA,
        pltpu.SemaphoreType.DMA,
    ],
)
def _som_distances(x_hbm, w_hbm, out_hbm, x_v, w_v, o_v, sem0, sem1):
    wid = lax.axis_index("s") * NC + lax.axis_index("c")
    q0 = wid * QPW
    cp0 = pltpu.async_copy(
        w_hbm.at[pl.ds(q0, QPW // 2)], w_v.at[pl.ds(0, QPW // 2)], sem0
    )
    cp1 = pltpu.async_copy(
        w_hbm.at[pl.ds(q0 + QPW // 2, QPW // 2)],
        w_v.at[pl.ds(QPW // 2, QPW // 2)],
        sem1,
    )
    pltpu.sync_copy(x_hbm, x_v)

    xs = [x_v[pl.ds(k * L, L)] for k in range(KD)]
    lanes = lax.iota(jnp.int32, L)
    masks = {s: (lanes & s) == 0 for s in (8, 4, 2, 1)}
    perms = {s: lanes ^ s for s in (8, 4, 2, 1)}

    def xperm(v, s):
        return v.at[perms[s]].get(mode="promise_in_bounds", unique_indices=True)

    def combine(a, b, s):
        return jnp.where(masks[s], a, xperm(b, s)) + jnp.where(
            masks[s], xperm(a, s), b
        )

    def acc_row(q, u):
        d = w_v[q, u, pl.ds(0, L)] - xs[0]
        acc = d * d
        for k in range(1, KD):
            d = w_v[q, u, pl.ds(k * L, L)] - xs[k]
            acc = acc + d * d
        return acc

    def group_body(g, carry):
        q = g >> 2
        c0 = (g & 3) << 4

        def quad(i):
            t8a = combine(
                acc_row(q, c0 + LEAF[4 * i]), acc_row(q, c0 + LEAF[4 * i + 1]), 8
            )
            t8b = combine(
                acc_row(q, c0 + LEAF[4 * i + 2]),
                acc_row(q, c0 + LEAF[4 * i + 3]),
                8,
            )
            return combine(t8a, t8b, 4)

        t2a = combine(quad(0), quad(1), 2)
        t2b = combine(quad(2), quad(3), 2)
        o_v[q, pl.ds(c0, L)] = combine(t2a, t2b, 1)
        return carry

    cp0.wait()
    lax.fori_loop(0, 8, group_body, 0)
    cp1.wait()
    lax.fori_loop(8, 16, group_body, 0)
    pltpu.sync_copy(o_v, out_hbm.at[pl.ds(q0, QPW)])


def kernel(x, weights):
    return _som_distances(x, weights)


# speedup vs baseline: 1.0868x; 1.0160x over previous
import functools

import jax
import jax.numpy as jnp
from jax import lax
from jax.experimental import pallas as pl
from jax.experimental.pallas import tpu as pltpu
from jax.experimental.pallas import tpu_sc as plsc

G0, G1, D = 128, 64, 256
L = 16
NC, NS = 2, 16
NW = NC * NS
QPW = G0 // NW
KD = D // L

LEAF = (0, 8, 4, 12, 2, 10, 6, 14, 1, 9, 5, 13, 3, 11, 7, 15)

_mesh = plsc.VectorSubcoreMesh(core_axis_name="c", subcore_axis_name="s")


@functools.partial(
    pl.kernel,
    mesh=_mesh,
    out_type=jax.ShapeDtypeStruct((G0, G1), jnp.float32),
    scratch_types=[
        pltpu.VMEM((D,), jnp.float32),
        pltpu.VMEM((QPW, G1, D), jnp.float32),
        pltpu.VMEM((QPW, G1), jnp.float32),
        pltpu.SemaphoreType.DMA,
        pltpu.SemaphoreType.DMA,
        pltpu.SemaphoreType.DMA,
        pltpu.SemaphoreType.DMA,
    ],
)
def _som_distances(x_hbm, w_hbm, out_hbm, x_v, w_v, o_v, s0, s1, s2, s3):
    wid = lax.axis_index("s") * NC + lax.axis_index("c")
    q0 = wid * QPW
    sems = (s0, s1, s2, s3)
    cps = [
        pltpu.async_copy(w_hbm.at[q0 + q], w_v.at[q], sems[q]) for q in range(QPW)
    ]
    pltpu.sync_copy(x_hbm, x_v)

    xs = [x_v[pl.ds(k * L, L)] for k in range(KD)]
    lanes = lax.iota(jnp.int32, L)
    masks = {s: (lanes & s) == 0 for s in (8, 4, 2, 1)}
    perms = {s: lanes ^ s for s in (8, 4, 2, 1)}

    def xperm(v, s):
        return v.at[perms[s]].get(mode="promise_in_bounds", unique_indices=True)

    def combine(a, b, s):
        return jnp.where(masks[s], a, xperm(b, s)) + jnp.where(
            masks[s], xperm(a, s), b
        )

    def acc_row(q, u):
        d = w_v[q, u, pl.ds(0, L)] - xs[0]
        acc = d * d
        for k in range(1, KD):
            d = w_v[q, u, pl.ds(k * L, L)] - xs[k]
            acc = acc + d * d
        return acc

    def group_body(g, carry):
        q = g >> 2
        c0 = (g & 3) << 4

        def quad(i):
            t8a = combine(
                acc_row(q, c0 + LEAF[4 * i]), acc_row(q, c0 + LEAF[4 * i + 1]), 8
            )
            t8b = combine(
                acc_row(q, c0 + LEAF[4 * i + 2]),
                acc_row(q, c0 + LEAF[4 * i + 3]),
                8,
            )
            return combine(t8a, t8b, 4)

        for b in (1, 2, 3):

            @pl.when(g == 4 * b)
            def _(b=b):
                cps[b].wait()

        t2a = combine(quad(0), quad(1), 2)
        t2b = combine(quad(2), quad(3), 2)
        o_v[q, pl.ds(c0, L)] = combine(t2a, t2b, 1)
        return carry

    cps[0].wait()
    lax.fori_loop(0, 16, group_body, 0)
    pltpu.sync_copy(o_v, out_hbm.at[pl.ds(q0, QPW)])


def kernel(x, weights):
    return _som_distances(x, weights)
